# baseline (speedup 1.0000x reference)
import functools

import jax
import jax.numpy as jnp
from jax import lax
from jax.experimental import pallas as pl
from jax.experimental.pallas import tpu as pltpu
from jax.experimental.pallas import tpu_sc as plsc

BATCH = 16384
EMBED = 64
NC = 2
NS = 16
NW = NC * NS
LANES = 16
BPW = BATCH // NW
CHUNK = 128
NCHUNK = BPW // CHUNK
NGROUP = BPW // LANES
EPS = 1e-12


def _rsqrt_nr(x):
    i = plsc.bitcast(x, jnp.int32)
    i = jnp.int32(0x5F3759DF) - (i >> 1)
    y = plsc.bitcast(i, jnp.float32)
    for _ in range(2):
        y = y * (1.5 - 0.5 * x * y * y)
    return y


def _body(user_hbm, item_hbm, uemb_hbm, iemb_hbm, out_hbm,
          uidx, iidx, urows, irows, outv, *sems):
    wid = lax.axis_index("s") * NC + lax.axis_index("c")
    base = wid * BPW

    pltpu.sync_copy(user_hbm.at[pl.ds(wid * NCHUNK, NCHUNK)], uidx)
    pltpu.sync_copy(item_hbm.at[pl.ds(wid * NCHUNK, NCHUNK)], iidx)

    copies = []
    for j in range(NCHUNK):
        copies.append((
            pltpu.async_copy(uemb_hbm.at[uidx.at[j]],
                             urows.at[pl.ds(j * CHUNK, CHUNK)], sems[2 * j]),
            pltpu.async_copy(iemb_hbm.at[iidx.at[j]],
                             irows.at[pl.ds(j * CHUNK, CHUNK)], sems[2 * j + 1]),
        ))

    iota = lax.iota(jnp.int32, LANES)

    def group(g, carry):
        rows = g * LANES + iota
        zero = jnp.zeros((LANES,), jnp.float32)
        su = [zero] * 4
        si = [zero] * 4
        dd = [zero] * 4
        for j in range(EMBED):
            cj = jnp.full((LANES,), j, jnp.int32)
            u = plsc.load_gather(urows, [rows, cj])
            v = plsc.load_gather(irows, [rows, cj])
            a = j & 3
            su[a] = su[a] + u * u
            si[a] = si[a] + v * v
            dd[a] = dd[a] + u * v
        su_t = (su[0] + su[1]) + (su[2] + su[3])
        si_t = (si[0] + si[1]) + (si[2] + si[3])
        dd_t = (dd[0] + dd[1]) + (dd[2] + dd[3])

        inv_u = jnp.where(su_t >= EPS * EPS, _rsqrt_nr(su_t), 1.0 / EPS)
        inv_i = jnp.where(si_t >= EPS * EPS, _rsqrt_nr(si_t), 1.0 / EPS)
        t = (su_t * inv_u * inv_u + si_t * inv_i * inv_i
             - 2.0 * dd_t * inv_u * inv_i)
        t = jnp.maximum(t, 0.0)
        dist = -(t * _rsqrt_nr(t))
        outv[pl.ds(g * LANES, LANES)] = dist
        return carry

    gpc = CHUNK // LANES
    for j in range(NCHUNK):
        cu, ci = copies[j]
        cu.wait()
        ci.wait()
        lax.fori_loop(j * gpc, (j + 1) * gpc, group, 0)

    pltpu.sync_copy(outv, out_hbm.at[pl.ds(base, BPW)])


@functools.partial(jax.jit, static_argnames=())
def _run(user2d, item2d, uemb, iemb):
    mesh = plsc.VectorSubcoreMesh(core_axis_name="c", subcore_axis_name="s")
    f = pl.kernel(
        _body,
        out_type=jax.ShapeDtypeStruct((BATCH,), jnp.float32),
        mesh=mesh,
        compiler_params=pltpu.CompilerParams(
            needs_layout_passes=False, use_tc_tiling_on_sc=False),
        scratch_types=[
            pltpu.VMEM((NCHUNK, CHUNK), jnp.int32),
            pltpu.VMEM((NCHUNK, CHUNK), jnp.int32),
            pltpu.VMEM((BPW, EMBED), jnp.float32),
            pltpu.VMEM((BPW, EMBED), jnp.float32),
            pltpu.VMEM((BPW,), jnp.float32),
        ] + [pltpu.SemaphoreType.DMA] * (2 * NCHUNK),
    )
    return f(user2d, item2d, uemb, iemb)


def kernel(user, item, k, user_embeddings, item_embeddings):
    del k
    user2d = user.astype(jnp.int32).reshape(NW * NCHUNK, CHUNK)
    item2d = item.astype(jnp.int32).reshape(NW * NCHUNK, CHUNK)
    return _run(user2d, item2d, user_embeddings, item_embeddings)
